# baseline (speedup 1.0000x reference)
import functools

import jax
import jax.numpy as jnp
from jax import lax
from jax.experimental import pallas as pl
from jax.experimental.pallas import tpu as pltpu
from jax.experimental.pallas import tpu_sc as plsc

_LAM1 = 3.0
_LAM2 = 3.0
_K = 3
_GAMMA = 1.0 / (1.0 + _LAM2)
_BETA = 1.0 / (2.0 * _GAMMA)

_N = 10000
_NPAD = 10240
_D = 128
_NTILES = 32
_C = 80

_MESH = plsc.VectorSubcoreMesh(core_axis_name="c", subcore_axis_name="s")


def _mlp_body(x_ref, w1_ref, b1_ref, w2_ref, b2_ref, o_ref):
    h = jnp.dot(x_ref[...], w1_ref[...], preferred_element_type=jnp.float32)
    h = jnp.maximum(h + b1_ref[...], 0.0)
    o = jnp.dot(h, w2_ref[...], preferred_element_type=jnp.float32)
    o_ref[...] = o + b2_ref[...]


def _mlp(x, W1, b1, W2, b2):
    n, d_in = x.shape
    hid = W1.shape[1]
    d_out = W2.shape[1]
    blk = 1000
    return pl.pallas_call(
        _mlp_body,
        grid=(n // blk,),
        in_specs=[
            pl.BlockSpec((blk, d_in), lambda i: (i, 0)),
            pl.BlockSpec((d_in, hid), lambda i: (0, 0)),
            pl.BlockSpec((1, hid), lambda i: (0, 0)),
            pl.BlockSpec((hid, d_out), lambda i: (0, 0)),
            pl.BlockSpec((1, d_out), lambda i: (0, 0)),
        ],
        out_specs=pl.BlockSpec((blk, d_out), lambda i: (i, 0)),
        out_shape=jax.ShapeDtypeStruct((n, d_out), jnp.float32),
    )(x, W1, b1.reshape(1, hid), W2, b2.reshape(1, d_out))


def _k0_body(p_ref, o_ref):
    deg = p_ref[0][:, :1] + p_ref[1][:, :1] + 1.0
    o_ref[...] = lax.rsqrt(deg)


def _deg_to_dinv(partials):
    blk = 1024
    return pl.pallas_call(
        _k0_body,
        grid=(_NPAD // blk,),
        in_specs=[pl.BlockSpec((2, blk, 16), lambda i: (0, i, 0))],
        out_specs=pl.BlockSpec((blk, 1), lambda i: (i, 0)),
        out_shape=jax.ShapeDtypeStruct((_NPAD, 1), jnp.float32),
    )(partials)


def _scale_body(d_ref, v_ref, o_ref):
    o_ref[...] = d_ref[...] * v_ref[...]


def _scale(dinv, v):
    blk = 1024
    return pl.pallas_call(
        _scale_body,
        grid=(_NPAD // blk,),
        in_specs=[
            pl.BlockSpec((blk, 1), lambda i: (i, 0)),
            pl.BlockSpec((blk, _D), lambda i: (i, 0)),
        ],
        out_specs=pl.BlockSpec((blk, _D), lambda i: (i, 0)),
        out_shape=jax.ShapeDtypeStruct((_NPAD, _D), jnp.float32),
    )(dinv, v)


def _k1_body(h_ref, a_ref, xk_ref, t_ref, d_ref, y_ref, xs2_ref):
    d = d_ref[...]
    adj = d * (a_ref[0] + a_ref[1]) + (d * d) * xk_ref[...]
    y = _GAMMA * h_ref[...] + (1.0 - _GAMMA) * adj
    y_ref[...] = y
    xs2_ref[...] = d * (y - _GAMMA * t_ref[...])


def _combine_y(h, acc, xk, tprev, dinv):
    blk = 1024
    return pl.pallas_call(
        _k1_body,
        grid=(_NPAD // blk,),
        in_specs=[
            pl.BlockSpec((blk, _D), lambda i: (i, 0)),
            pl.BlockSpec((2, blk, _D), lambda i: (0, i, 0)),
            pl.BlockSpec((blk, _D), lambda i: (i, 0)),
            pl.BlockSpec((blk, _D), lambda i: (i, 0)),
            pl.BlockSpec((blk, 1), lambda i: (i, 0)),
        ],
        out_specs=[
            pl.BlockSpec((blk, _D), lambda i: (i, 0)),
            pl.BlockSpec((blk, _D), lambda i: (i, 0)),
        ],
        out_shape=[
            jax.ShapeDtypeStruct((_NPAD, _D), jnp.float32),
            jax.ShapeDtypeStruct((_NPAD, _D), jnp.float32),
        ],
    )(h, acc, xk, tprev, dinv)


def _k2_body(y_ref, t_ref, d_ref, xk_ref, t_out_ref):
    d = d_ref[...]
    t = d * (t_ref[0] + t_ref[1])
    t_out_ref[...] = t
    xk_ref[...] = y_ref[...] - _GAMMA * t


def _combine_xk(y, tpart, dinv):
    blk = 1024
    return pl.pallas_call(
        _k2_body,
        grid=(_NPAD // blk,),
        in_specs=[
            pl.BlockSpec((blk, _D), lambda i: (i, 0)),
            pl.BlockSpec((2, blk, _D), lambda i: (0, i, 0)),
            pl.BlockSpec((blk, 1), lambda i: (i, 0)),
        ],
        out_specs=[
            pl.BlockSpec((blk, _D), lambda i: (i, 0)),
            pl.BlockSpec((blk, _D), lambda i: (i, 0)),
        ],
        out_shape=[
            jax.ShapeDtypeStruct((_NPAD, _D), jnp.float32),
            jax.ShapeDtypeStruct((_NPAD, _D), jnp.float32),
        ],
    )(y, tpart, dinv)


def _mask_body(r_ref, c_ref, rm_ref, cm_ref):
    r = r_ref[...]
    c = c_ref[...]
    m = r > c
    dummy = jnp.full_like(r, _N)
    rm_ref[...] = jnp.where(m, r, dummy)
    cm_ref[...] = jnp.where(m, c, dummy)


def _mask_idx(row, col, e_total):
    cols = 512
    rows = e_total // cols
    rm, cm = pl.pallas_call(
        _mask_body,
        grid=(1,),
        in_specs=[
            pl.BlockSpec((rows, cols), lambda i: (0, 0)),
            pl.BlockSpec((rows, cols), lambda i: (0, 0)),
        ],
        out_specs=[
            pl.BlockSpec((rows, cols), lambda i: (0, 0)),
            pl.BlockSpec((rows, cols), lambda i: (0, 0)),
        ],
        out_shape=[
            jax.ShapeDtypeStruct((rows, cols), jnp.int32),
            jax.ShapeDtypeStruct((rows, cols), jnp.int32),
        ],
    )(row.reshape(rows, cols), col.reshape(rows, cols))
    return rm.reshape(-1), cm.reshape(-1)


def _zup_body(z_ref, gr_ref, gc_ref, zn_ref, zneg_ref):
    zb = z_ref[...] + _BETA * (gr_ref[...] - gc_ref[...])
    rn2 = jnp.sum(zb * zb, axis=1, keepdims=True)
    scale = jnp.minimum(jnp.float32(1.0), _LAM1 * lax.rsqrt(rn2))
    zn = scale * zb
    zn_ref[...] = zn
    zneg_ref[...] = -zn


def _zupdate(z, gr, gc, e_total):
    blk = 4000
    return pl.pallas_call(
        _zup_body,
        grid=(e_total // blk,),
        in_specs=[
            pl.BlockSpec((blk, _D), lambda i: (i, 0)),
            pl.BlockSpec((blk, _D), lambda i: (i, 0)),
            pl.BlockSpec((blk, _D), lambda i: (i, 0)),
        ],
        out_specs=[
            pl.BlockSpec((blk, _D), lambda i: (i, 0)),
            pl.BlockSpec((blk, _D), lambda i: (i, 0)),
        ],
        out_shape=[
            jax.ShapeDtypeStruct((e_total, _D), jnp.float32),
            jax.ShapeDtypeStruct((e_total, _D), jnp.float32),
        ],
    )(z, gr, gc)


def _zero_vmem_block(buf):
    z16 = jnp.zeros((16,), jnp.float32)

    def body(e, carry):
        for dd in range(8):
            buf[e, pl.ds(dd * 16, 16)] = z16
        return carry

    lax.fori_loop(0, _C, body, 0)


def _zero_shared(S, zbuf, s):
    def body(i, carry):
        S_slice = S.at[pl.ds(s * 640 + i * _C, _C)]
        pltpu.sync_copy(zbuf, S_slice)
        return carry

    lax.fori_loop(0, 640 // _C, body, 0)


def _writeback_shared(S, out_at_c, s):
    pltpu.sync_copy(S.at[pl.ds(s * 640, 640)], out_at_c.at[pl.ds(s * 640, 640)])


def _edges_per_tile(e_total):
    return e_total // _NTILES


def _deg_kernel(e_total):
    ept = _edges_per_tile(e_total)
    nchunk = ept // _C

    @functools.partial(
        pl.kernel,
        mesh=_MESH,
        out_type=jax.ShapeDtypeStruct((2, _NPAD, 16), jnp.float32),
        scratch_types=[
            pltpu.VMEM((_C,), jnp.int32),
            pltpu.VMEM((_C, 16), jnp.float32),
            pltpu.VMEM((_C, 16), jnp.float32),
            pltpu.VMEM_SHARED((_NPAD, 16), jnp.float32),
        ],
    )
    def k(row_hbm, out_hbm, idxb, onesb, zb, S):
        c = lax.axis_index("c")
        s = lax.axis_index("s")
        wid = c * 16 + s

        z16 = jnp.zeros((16,), jnp.float32)
        o16 = jnp.ones((16,), jnp.float32)

        def fill(e, carry):
            onesb[e] = o16
            zb[e] = z16
            return carry

        lax.fori_loop(0, _C, fill, 0)

        def zs(i, carry):
            pltpu.sync_copy(zb, S.at[pl.ds(s * 640 + i * _C, _C)])
            return carry

        lax.fori_loop(0, 640 // _C, zs, 0)
        plsc.subcore_barrier()

        def chunk(j, carry):
            base = wid * ept + j * _C
            pltpu.sync_copy(row_hbm.at[pl.ds(base, _C)], idxb)
            pltpu.sync_copy(onesb, S.at[idxb], add=True)
            return carry

        lax.fori_loop(0, nchunk, chunk, 0)
        plsc.subcore_barrier()
        pltpu.sync_copy(
            S.at[pl.ds(s * 640, 640)], out_hbm.at[c].at[pl.ds(s * 640, 640)]
        )

    return k


def _adj_kernel(e_total):
    ept = _edges_per_tile(e_total)
    nchunk = ept // _C

    @functools.partial(
        pl.kernel,
        mesh=_MESH,
        out_type=jax.ShapeDtypeStruct((2, _NPAD, _D), jnp.float32),
        scratch_types=[
            pltpu.VMEM((_C,), jnp.int32),
            pltpu.VMEM((_C,), jnp.int32),
            pltpu.VMEM((_C, _D), jnp.float32),
            pltpu.VMEM((_C, _D), jnp.float32),
            pltpu.VMEM_SHARED((_NPAD, _D), jnp.float32),
            pltpu.SemaphoreType.DMA,
        ],
    )
    def k(xs_hbm, row_hbm, col_hbm, out_hbm, rowb, colb, gbuf, zbuf, S, sem):
        c = lax.axis_index("c")
        s = lax.axis_index("s")
        wid = c * 16 + s

        _zero_vmem_block(zbuf)
        _zero_shared(S, zbuf, s)
        plsc.subcore_barrier()

        def chunk(j, carry):
            base = wid * ept + j * _C
            pltpu.sync_copy(row_hbm.at[pl.ds(base, _C)], rowb)
            pltpu.sync_copy(col_hbm.at[pl.ds(base, _C)], colb)
            pltpu.async_copy(xs_hbm.at[colb], gbuf, sem).wait()
            pltpu.sync_copy(gbuf, S.at[rowb], add=True)
            return carry

        lax.fori_loop(0, nchunk, chunk, 0)
        plsc.subcore_barrier()
        _writeback_shared(S, out_hbm.at[c], s)

    return k


def _gather_kernel(e_total):
    ept = _edges_per_tile(e_total)
    nchunk = ept // _C

    @functools.partial(
        pl.kernel,
        mesh=_MESH,
        out_type=[
            jax.ShapeDtypeStruct((e_total, _D), jnp.float32),
            jax.ShapeDtypeStruct((e_total, _D), jnp.float32),
        ],
        scratch_types=[
            pltpu.VMEM((_C,), jnp.int32),
            pltpu.VMEM((_C,), jnp.int32),
            pltpu.VMEM((_C, _D), jnp.float32),
            pltpu.VMEM((_C, _D), jnp.float32),
            pltpu.SemaphoreType.DMA,
            pltpu.SemaphoreType.DMA,
        ],
    )
    def k(xs2_hbm, rowm_hbm, colm_hbm, gr_hbm, gc_hbm,
          rowb, colb, gr, gc, sem1, sem2):
        c = lax.axis_index("c")
        s = lax.axis_index("s")
        wid = c * 16 + s

        def chunk(j, carry):
            base = wid * ept + j * _C
            pltpu.sync_copy(rowm_hbm.at[pl.ds(base, _C)], rowb)
            pltpu.sync_copy(colm_hbm.at[pl.ds(base, _C)], colb)
            pltpu.async_copy(xs2_hbm.at[rowb], gr, sem1).wait()
            pltpu.async_copy(xs2_hbm.at[colb], gc, sem2).wait()
            pltpu.sync_copy(gr, gr_hbm.at[pl.ds(base, _C)])
            pltpu.sync_copy(gc, gc_hbm.at[pl.ds(base, _C)])
            return carry

        lax.fori_loop(0, nchunk, chunk, 0)

    return k


def _tpass_kernel(e_total):
    ept = _edges_per_tile(e_total)
    nchunk = ept // _C

    @functools.partial(
        pl.kernel,
        mesh=_MESH,
        out_type=jax.ShapeDtypeStruct((2, _NPAD, _D), jnp.float32),
        scratch_types=[
            pltpu.VMEM((_C,), jnp.int32),
            pltpu.VMEM((_C,), jnp.int32),
            pltpu.VMEM((_C, _D), jnp.float32),
            pltpu.VMEM((_C, _D), jnp.float32),
            pltpu.VMEM((_C, _D), jnp.float32),
            pltpu.VMEM_SHARED((_NPAD, _D), jnp.float32),
        ],
    )
    def k(zp_hbm, zn_hbm, row_hbm, col_hbm, out_hbm, rowb, colb, bufp, bufn,
          zbuf, S):
        c = lax.axis_index("c")
        s = lax.axis_index("s")
        wid = c * 16 + s

        _zero_vmem_block(zbuf)
        _zero_shared(S, zbuf, s)
        plsc.subcore_barrier()

        def chunk(j, carry):
            base = wid * ept + j * _C
            pltpu.sync_copy(row_hbm.at[pl.ds(base, _C)], rowb)
            pltpu.sync_copy(col_hbm.at[pl.ds(base, _C)], colb)
            pltpu.sync_copy(zp_hbm.at[pl.ds(base, _C)], bufp)
            pltpu.sync_copy(zn_hbm.at[pl.ds(base, _C)], bufn)
            pltpu.sync_copy(bufp, S.at[rowb], add=True)
            pltpu.sync_copy(bufn, S.at[colb], add=True)
            return carry

        lax.fori_loop(0, nchunk, chunk, 0)
        plsc.subcore_barrier()
        _writeback_shared(S, out_hbm.at[c], s)

    return k


def kernel(x, edge_index, W1, b1, W2, b2):
    n = x.shape[0]
    e_total = edge_index.shape[1]
    row = edge_index[0]
    col = edge_index[1]

    h = _mlp(x, W1, b1, W2, b2)
    h = jnp.pad(h, ((0, _NPAD - n), (0, 0)))

    deg_partials = _deg_kernel(e_total)(row)
    dinv = _deg_to_dinv(deg_partials)
    rowm, colm = _mask_idx(row, col, e_total)

    adj_pass = _adj_kernel(e_total)
    gather_pass = _gather_kernel(e_total)
    t_pass = _tpass_kernel(e_total)

    xk = h
    xs = _scale(dinv, h)
    tprev = jnp.zeros((_NPAD, _D), jnp.float32)
    z = jnp.zeros((e_total, _D), jnp.float32)

    for it in range(_K):
        acc = adj_pass(xs, row, col)
        y, xs2 = _combine_y(h, acc, xk, tprev, dinv)
        gr, gc = gather_pass(xs2, rowm, colm)
        z, zneg = _zupdate(z, gr, gc, e_total)
        tpart = t_pass(z, zneg, row, col)
        xk, tprev = _combine_xk(y, tpart, dinv)
        if it + 1 < _K:
            xs = _scale(dinv, xk)
    return xk[:n]


# speedup vs baseline: 8.2837x; 8.2837x over previous
import functools

import jax
import jax.numpy as jnp
from jax import lax
from jax.experimental import pallas as pl
from jax.experimental.pallas import tpu as pltpu
from jax.experimental.pallas import tpu_sc as plsc

_LAM1 = 3.0
_LAM2 = 3.0
_K = 3
_GAMMA = 1.0 / (1.0 + _LAM2)
_BETA = 1.0 / (2.0 * _GAMMA)

_N = 10000
_NPAD = 10240
_D = 128
_NTILES = 32
_C = 80

_MESH = plsc.VectorSubcoreMesh(core_axis_name="c", subcore_axis_name="s")


def _mlp_body(x_ref, w1_ref, b1_ref, w2_ref, b2_ref, o_ref):
    h = jnp.dot(x_ref[...], w1_ref[...], preferred_element_type=jnp.float32)
    h = jnp.maximum(h + b1_ref[...], 0.0)
    o = jnp.dot(h, w2_ref[...], preferred_element_type=jnp.float32)
    o_ref[...] = o + b2_ref[...]


def _mlp(x, W1, b1, W2, b2):
    n, d_in = x.shape
    hid = W1.shape[1]
    d_out = W2.shape[1]
    blk = 1000
    return pl.pallas_call(
        _mlp_body,
        grid=(n // blk,),
        in_specs=[
            pl.BlockSpec((blk, d_in), lambda i: (i, 0)),
            pl.BlockSpec((d_in, hid), lambda i: (0, 0)),
            pl.BlockSpec((1, hid), lambda i: (0, 0)),
            pl.BlockSpec((hid, d_out), lambda i: (0, 0)),
            pl.BlockSpec((1, d_out), lambda i: (0, 0)),
        ],
        out_specs=pl.BlockSpec((blk, d_out), lambda i: (i, 0)),
        out_shape=jax.ShapeDtypeStruct((n, d_out), jnp.float32),
    )(x, W1, b1.reshape(1, hid), W2, b2.reshape(1, d_out))


def _k0_body(p_ref, o_ref):
    deg = p_ref[0][:, :1] + p_ref[1][:, :1] + 1.0
    o_ref[...] = lax.rsqrt(deg)


def _deg_to_dinv(partials):
    blk = 1024
    return pl.pallas_call(
        _k0_body,
        grid=(_NPAD // blk,),
        in_specs=[pl.BlockSpec((2, blk, 16), lambda i: (0, i, 0))],
        out_specs=pl.BlockSpec((blk, 1), lambda i: (i, 0)),
        out_shape=jax.ShapeDtypeStruct((_NPAD, 1), jnp.float32),
    )(partials)


def _scale_body(d_ref, v_ref, o_ref):
    o_ref[...] = d_ref[...] * v_ref[...]


def _scale(dinv, v):
    blk = 1024
    return pl.pallas_call(
        _scale_body,
        grid=(_NPAD // blk,),
        in_specs=[
            pl.BlockSpec((blk, 1), lambda i: (i, 0)),
            pl.BlockSpec((blk, _D), lambda i: (i, 0)),
        ],
        out_specs=pl.BlockSpec((blk, _D), lambda i: (i, 0)),
        out_shape=jax.ShapeDtypeStruct((_NPAD, _D), jnp.float32),
    )(dinv, v)


def _k1_body(h_ref, a_ref, xk_ref, t_ref, d_ref, y_ref, xs2_ref):
    d = d_ref[...]
    adj = d * (a_ref[0] + a_ref[1]) + (d * d) * xk_ref[...]
    y = _GAMMA * h_ref[...] + (1.0 - _GAMMA) * adj
    y_ref[...] = y
    xs2_ref[...] = d * (y - _GAMMA * t_ref[...])


def _combine_y(h, acc, xk, tprev, dinv):
    blk = 1024
    return pl.pallas_call(
        _k1_body,
        grid=(_NPAD // blk,),
        in_specs=[
            pl.BlockSpec((blk, _D), lambda i: (i, 0)),
            pl.BlockSpec((2, blk, _D), lambda i: (0, i, 0)),
            pl.BlockSpec((blk, _D), lambda i: (i, 0)),
            pl.BlockSpec((blk, _D), lambda i: (i, 0)),
            pl.BlockSpec((blk, 1), lambda i: (i, 0)),
        ],
        out_specs=[
            pl.BlockSpec((blk, _D), lambda i: (i, 0)),
            pl.BlockSpec((blk, _D), lambda i: (i, 0)),
        ],
        out_shape=[
            jax.ShapeDtypeStruct((_NPAD, _D), jnp.float32),
            jax.ShapeDtypeStruct((_NPAD, _D), jnp.float32),
        ],
    )(h, acc, xk, tprev, dinv)


def _k2_body(y_ref, t_ref, d_ref, xk_ref, t_out_ref):
    d = d_ref[...]
    t = d * (t_ref[0] + t_ref[1])
    t_out_ref[...] = t
    xk_ref[...] = y_ref[...] - _GAMMA * t


def _combine_xk(y, tpart, dinv):
    blk = 1024
    return pl.pallas_call(
        _k2_body,
        grid=(_NPAD // blk,),
        in_specs=[
            pl.BlockSpec((blk, _D), lambda i: (i, 0)),
            pl.BlockSpec((2, blk, _D), lambda i: (0, i, 0)),
            pl.BlockSpec((blk, 1), lambda i: (i, 0)),
        ],
        out_specs=[
            pl.BlockSpec((blk, _D), lambda i: (i, 0)),
            pl.BlockSpec((blk, _D), lambda i: (i, 0)),
        ],
        out_shape=[
            jax.ShapeDtypeStruct((_NPAD, _D), jnp.float32),
            jax.ShapeDtypeStruct((_NPAD, _D), jnp.float32),
        ],
    )(y, tpart, dinv)


def _mask_body(r_ref, c_ref, rm_ref, cm_ref):
    r = r_ref[...]
    c = c_ref[...]
    m = r > c
    rm_ref[...] = r
    cm_ref[...] = jnp.where(m, c, r)


def _mask_idx(row, col, e_total):
    cols = 512
    rows = e_total // cols
    rm, cm = pl.pallas_call(
        _mask_body,
        grid=(1,),
        in_specs=[
            pl.BlockSpec((rows, cols), lambda i: (0, 0)),
            pl.BlockSpec((rows, cols), lambda i: (0, 0)),
        ],
        out_specs=[
            pl.BlockSpec((rows, cols), lambda i: (0, 0)),
            pl.BlockSpec((rows, cols), lambda i: (0, 0)),
        ],
        out_shape=[
            jax.ShapeDtypeStruct((rows, cols), jnp.int32),
            jax.ShapeDtypeStruct((rows, cols), jnp.int32),
        ],
    )(row.reshape(rows, cols), col.reshape(rows, cols))
    return rm.reshape(-1), cm.reshape(-1)


def _zup_body(z_ref, gr_ref, gc_ref, zn_ref, zneg_ref):
    zb = z_ref[...] + _BETA * (gr_ref[...] - gc_ref[...])
    rn2 = jnp.sum(zb * zb, axis=1, keepdims=True)
    scale = jnp.minimum(jnp.float32(1.0), _LAM1 * lax.rsqrt(rn2))
    zn = scale * zb
    zn_ref[...] = zn
    zneg_ref[...] = -zn


def _zupdate(z, gr, gc, e_total):
    blk = 4000
    return pl.pallas_call(
        _zup_body,
        grid=(e_total // blk,),
        in_specs=[
            pl.BlockSpec((blk, _D), lambda i: (i, 0)),
            pl.BlockSpec((blk, _D), lambda i: (i, 0)),
            pl.BlockSpec((blk, _D), lambda i: (i, 0)),
        ],
        out_specs=[
            pl.BlockSpec((blk, _D), lambda i: (i, 0)),
            pl.BlockSpec((blk, _D), lambda i: (i, 0)),
        ],
        out_shape=[
            jax.ShapeDtypeStruct((e_total, _D), jnp.float32),
            jax.ShapeDtypeStruct((e_total, _D), jnp.float32),
        ],
    )(z, gr, gc)


def _zero_vmem_block(buf):
    z16 = jnp.zeros((16,), jnp.float32)

    def body(e, carry):
        for dd in range(8):
            buf[e, pl.ds(dd * 16, 16)] = z16
        return carry

    lax.fori_loop(0, _C, body, 0)


def _zero_shared(S, zbuf, s):
    def body(i, carry):
        S_slice = S.at[pl.ds(s * 640 + i * _C, _C)]
        pltpu.sync_copy(zbuf, S_slice)
        return carry

    lax.fori_loop(0, 640 // _C, body, 0)


def _writeback_shared(S, out_at_c, s):
    pltpu.sync_copy(S.at[pl.ds(s * 640, 640)], out_at_c.at[pl.ds(s * 640, 640)])


def _edges_per_tile(e_total):
    return e_total // _NTILES


def _deg_kernel(e_total):
    ept = _edges_per_tile(e_total)
    nchunk = ept // _C

    @functools.partial(
        pl.kernel,
        mesh=_MESH,
        out_type=jax.ShapeDtypeStruct((2, _NPAD, 16), jnp.float32),
        scratch_types=[
            pltpu.VMEM((_C,), jnp.int32),
            pltpu.VMEM((_C, 16), jnp.float32),
            pltpu.VMEM((_C, 16), jnp.float32),
            pltpu.VMEM_SHARED((_NPAD, 16), jnp.float32),
        ],
    )
    def k(row_hbm, out_hbm, idxb, onesb, zb, S):
        c = lax.axis_index("c")
        s = lax.axis_index("s")
        wid = c * 16 + s

        z16 = jnp.zeros((16,), jnp.float32)
        o16 = jnp.ones((16,), jnp.float32)

        def fill(e, carry):
            onesb[e] = o16
            zb[e] = z16
            return carry

        lax.fori_loop(0, _C, fill, 0)

        def zs(i, carry):
            pltpu.sync_copy(zb, S.at[pl.ds(s * 640 + i * _C, _C)])
            return carry

        lax.fori_loop(0, 640 // _C, zs, 0)
        plsc.subcore_barrier()

        def chunk(j, carry):
            base = wid * ept + j * _C
            pltpu.sync_copy(row_hbm.at[pl.ds(base, _C)], idxb)
            pltpu.sync_copy(onesb, S.at[idxb], add=True)
            return carry

        lax.fori_loop(0, nchunk, chunk, 0)
        plsc.subcore_barrier()
        pltpu.sync_copy(
            S.at[pl.ds(s * 640, 640)], out_hbm.at[c].at[pl.ds(s * 640, 640)]
        )

    return k


def _adj_kernel(e_total):
    ept = _edges_per_tile(e_total)
    nchunk = ept // _C

    @functools.partial(
        pl.kernel,
        mesh=_MESH,
        out_type=jax.ShapeDtypeStruct((2, _NPAD, _D), jnp.float32),
        scratch_types=[
            pltpu.VMEM((_C,), jnp.int32),
            pltpu.VMEM((_C,), jnp.int32),
            pltpu.VMEM((_C, _D), jnp.float32),
            pltpu.VMEM((_C, _D), jnp.float32),
            pltpu.VMEM_SHARED((_NPAD, _D), jnp.float32),
            pltpu.SemaphoreType.DMA,
        ],
    )
    def k(xs_hbm, row_hbm, col_hbm, out_hbm, rowb, colb, gbuf, zbuf, S, sem):
        c = lax.axis_index("c")
        s = lax.axis_index("s")
        wid = c * 16 + s

        _zero_vmem_block(zbuf)
        _zero_shared(S, zbuf, s)
        plsc.subcore_barrier()

        def chunk(j, carry):
            base = wid * ept + j * _C
            pltpu.sync_copy(row_hbm.at[pl.ds(base, _C)], rowb)
            pltpu.sync_copy(col_hbm.at[pl.ds(base, _C)], colb)
            pltpu.async_copy(xs_hbm.at[colb], gbuf, sem).wait()
            pltpu.sync_copy(gbuf, S.at[rowb], add=True)
            return carry

        lax.fori_loop(0, nchunk, chunk, 0)
        plsc.subcore_barrier()
        _writeback_shared(S, out_hbm.at[c], s)

    return k


def _gather_kernel(e_total):
    ept = _edges_per_tile(e_total)
    nchunk = ept // _C

    @functools.partial(
        pl.kernel,
        mesh=_MESH,
        out_type=[
            jax.ShapeDtypeStruct((e_total, _D), jnp.float32),
            jax.ShapeDtypeStruct((e_total, _D), jnp.float32),
        ],
        scratch_types=[
            pltpu.VMEM((_C,), jnp.int32),
            pltpu.VMEM((_C,), jnp.int32),
            pltpu.VMEM((_C, _D), jnp.float32),
            pltpu.VMEM((_C, _D), jnp.float32),
            pltpu.SemaphoreType.DMA,
            pltpu.SemaphoreType.DMA,
        ],
    )
    def k(xs2_hbm, rowm_hbm, colm_hbm, gr_hbm, gc_hbm,
          rowb, colb, gr, gc, sem1, sem2):
        c = lax.axis_index("c")
        s = lax.axis_index("s")
        wid = c * 16 + s

        def chunk(j, carry):
            base = wid * ept + j * _C
            pltpu.sync_copy(rowm_hbm.at[pl.ds(base, _C)], rowb)
            pltpu.sync_copy(colm_hbm.at[pl.ds(base, _C)], colb)
            pltpu.async_copy(xs2_hbm.at[rowb], gr, sem1).wait()
            pltpu.async_copy(xs2_hbm.at[colb], gc, sem2).wait()
            pltpu.sync_copy(gr, gr_hbm.at[pl.ds(base, _C)])
            pltpu.sync_copy(gc, gc_hbm.at[pl.ds(base, _C)])
            return carry

        lax.fori_loop(0, nchunk, chunk, 0)

    return k


def _tpass_kernel(e_total):
    ept = _edges_per_tile(e_total)
    nchunk = ept // _C

    @functools.partial(
        pl.kernel,
        mesh=_MESH,
        out_type=jax.ShapeDtypeStruct((2, _NPAD, _D), jnp.float32),
        scratch_types=[
            pltpu.VMEM((_C,), jnp.int32),
            pltpu.VMEM((_C,), jnp.int32),
            pltpu.VMEM((_C, _D), jnp.float32),
            pltpu.VMEM((_C, _D), jnp.float32),
            pltpu.VMEM((_C, _D), jnp.float32),
            pltpu.VMEM_SHARED((_NPAD, _D), jnp.float32),
        ],
    )
    def k(zp_hbm, zn_hbm, row_hbm, col_hbm, out_hbm, rowb, colb, bufp, bufn,
          zbuf, S):
        c = lax.axis_index("c")
        s = lax.axis_index("s")
        wid = c * 16 + s

        _zero_vmem_block(zbuf)
        _zero_shared(S, zbuf, s)
        plsc.subcore_barrier()

        def chunk(j, carry):
            base = wid * ept + j * _C
            pltpu.sync_copy(row_hbm.at[pl.ds(base, _C)], rowb)
            pltpu.sync_copy(col_hbm.at[pl.ds(base, _C)], colb)
            pltpu.sync_copy(zp_hbm.at[pl.ds(base, _C)], bufp)
            pltpu.sync_copy(zn_hbm.at[pl.ds(base, _C)], bufn)
            pltpu.sync_copy(bufp, S.at[rowb], add=True)
            pltpu.sync_copy(bufn, S.at[colb], add=True)
            return carry

        lax.fori_loop(0, nchunk, chunk, 0)
        plsc.subcore_barrier()
        _writeback_shared(S, out_hbm.at[c], s)

    return k


def kernel(x, edge_index, W1, b1, W2, b2):
    n = x.shape[0]
    e_total = edge_index.shape[1]
    row = edge_index[0]
    col = edge_index[1]

    h = _mlp(x, W1, b1, W2, b2)
    h = jnp.pad(h, ((0, _NPAD - n), (0, 0)))

    deg_partials = _deg_kernel(e_total)(row)
    dinv = _deg_to_dinv(deg_partials)
    rowm, colm = _mask_idx(row, col, e_total)

    adj_pass = _adj_kernel(e_total)
    gather_pass = _gather_kernel(e_total)
    t_pass = _tpass_kernel(e_total)

    xk = h
    xs = _scale(dinv, h)
    tprev = jnp.zeros((_NPAD, _D), jnp.float32)
    z = jnp.zeros((e_total, _D), jnp.float32)

    for it in range(_K):
        acc = adj_pass(xs, row, col)
        y, xs2 = _combine_y(h, acc, xk, tprev, dinv)
        gr, gc = gather_pass(xs2, rowm, colm)
        z, zneg = _zupdate(z, gr, gc, e_total)
        tpart = t_pass(z, zneg, row, col)
        xk, tprev = _combine_xk(y, tpart, dinv)
        if it + 1 < _K:
            xs = _scale(dinv, xk)
    return xk[:n]
